# baseline (speedup 1.0000x reference)
import jax
import jax.numpy as jnp
from jax import lax
from jax.experimental import pallas as pl
from jax.experimental.pallas import tpu as pltpu

LEN_A = 2000
LEN_B = 1500
SEQ = LEN_A + LEN_B
PADDED = 3584
PAD = PADDED - SEQ
D = 1024
BATCH = 8
BLK = 128
NBLK = PADDED // BLK


def _pack_body(a_ref, b_ref, out_ref, zeros_ref, sem):
    zeros_ref[...] = jnp.zeros_like(zeros_ref)
    copies = []
    for i in range(BATCH):
        copies.append(pltpu.make_async_copy(
            a_ref.at[pl.ds(i * LEN_A * D, LEN_A * D)],
            out_ref.at[pl.ds(i * PADDED * D, LEN_A * D)], sem))
        copies.append(pltpu.make_async_copy(
            b_ref.at[pl.ds(i * LEN_B * D, LEN_B * D)],
            out_ref.at[pl.ds((i * PADDED + LEN_A) * D, LEN_B * D)], sem))
        copies.append(pltpu.make_async_copy(
            zeros_ref,
            out_ref.at[pl.ds((i * PADDED + SEQ) * D, PAD * D)], sem))
    for c in copies:
        c.start()
    for c in copies:
        c.wait()


def _mask_body(mask_ref, ids_ref):
    i = pl.program_id(0)
    row = i * BLK + lax.broadcasted_iota(jnp.int32, (BLK, PADDED), 0)
    col = lax.broadcasted_iota(jnp.int32, (BLK, PADDED), 1)
    in_a = (row < LEN_A) & (col < LEN_A)
    in_b = (row >= LEN_A) & (row < SEQ) & (col >= LEN_A) & (col < SEQ)
    m = (col <= row) & (in_a | in_b)
    mask_ref[...] = m.reshape(1, 1, BLK, PADDED)

    @pl.when(i == 0)
    def _():
        c = lax.broadcasted_iota(jnp.int32, (1, PADDED), 1)
        ids_ref[...] = jnp.where(c < LEN_A, 0, jnp.where(c < SEQ, 1, -1))


@jax.jit
def kernel(seq_a, seq_b):
    packed = pl.pallas_call(
        _pack_body,
        out_shape=jax.ShapeDtypeStruct((BATCH * PADDED * D,), jnp.float32),
        in_specs=[
            pl.BlockSpec(memory_space=pl.ANY),
            pl.BlockSpec(memory_space=pl.ANY),
        ],
        out_specs=pl.BlockSpec(memory_space=pl.ANY),
        scratch_shapes=[
            pltpu.VMEM((PAD * D,), jnp.float32),
            pltpu.SemaphoreType.DMA,
        ],
    )(seq_a.reshape(-1), seq_b.reshape(-1))
    packed = packed.reshape(BATCH, PADDED, D)

    mask, ids = pl.pallas_call(
        _mask_body,
        grid=(NBLK,),
        out_shape=[
            jax.ShapeDtypeStruct((1, 1, PADDED, PADDED), jnp.bool_),
            jax.ShapeDtypeStruct((1, PADDED), jnp.int32),
        ],
        out_specs=[
            pl.BlockSpec((1, 1, BLK, PADDED), lambda i: (0, 0, i, 0)),
            pl.BlockSpec((1, PADDED), lambda i: (0, 0)),
        ],
    )()

    return packed, mask, ids.reshape(PADDED)


# speedup vs baseline: 1.0602x; 1.0602x over previous
import jax
import jax.numpy as jnp
from jax import lax
from jax.experimental import pallas as pl
from jax.experimental.pallas import tpu as pltpu

LEN_A = 2000
LEN_B = 1500
SEQ = LEN_A + LEN_B
PADDED = 3584
PAD = PADDED - SEQ
D = 1024
BATCH = 8
BLK = 128
NBLK = PADDED // BLK

B_MAIN = (LEN_B // 8) * 8
B_TAIL = LEN_B - B_MAIN
STAGE = PADDED - LEN_A - B_MAIN


def _pack_body(a_ref, b_ref, out_ref, tail_vm, stage_vm, sem_a, sem_b, sem_t,
               sem_s):
    cp_a = pltpu.make_async_copy(a_ref, out_ref.at[:, 0:LEN_A, :], sem_a)
    cp_b = pltpu.make_async_copy(b_ref.at[:, 0:B_MAIN, :],
                                 out_ref.at[:, LEN_A:LEN_A + B_MAIN, :], sem_b)
    t_start = pl.multiple_of(B_MAIN + pl.program_id(0), 8)
    cp_t = pltpu.make_async_copy(b_ref.at[:, pl.ds(t_start, 8), :], tail_vm,
                                 sem_t)
    cp_a.start()
    cp_b.start()
    cp_t.start()

    cp_t.wait()
    stage_vm[...] = jnp.zeros((BATCH, STAGE, D), jnp.float32)
    stage_vm[:, 0:B_TAIL, :] = tail_vm[:, 0:B_TAIL, :]
    cp_s = pltpu.make_async_copy(stage_vm,
                                 out_ref.at[:, LEN_A + B_MAIN:PADDED, :],
                                 sem_s)
    cp_s.start()

    cp_a.wait()
    cp_b.wait()
    cp_s.wait()


def _mask_body(mask_ref, ids_ref):
    i = pl.program_id(0)
    row = i * BLK + lax.broadcasted_iota(jnp.int32, (BLK, PADDED), 0)
    col = lax.broadcasted_iota(jnp.int32, (BLK, PADDED), 1)
    in_a = (row < LEN_A) & (col < LEN_A)
    in_b = (row >= LEN_A) & (row < SEQ) & (col >= LEN_A) & (col < SEQ)
    m = (col <= row) & (in_a | in_b)
    mask_ref[...] = m.reshape(1, 1, BLK, PADDED)

    @pl.when(i == 0)
    def _():
        c = lax.broadcasted_iota(jnp.int32, (1, PADDED), 1)
        ids_ref[...] = jnp.where(c < LEN_A, 0, jnp.where(c < SEQ, 1, -1))


@jax.jit
def kernel(seq_a, seq_b):
    packed = pl.pallas_call(
        _pack_body,
        grid=(1,),
        out_shape=jax.ShapeDtypeStruct((BATCH, PADDED, D), jnp.float32),
        in_specs=[
            pl.BlockSpec(memory_space=pl.ANY),
            pl.BlockSpec(memory_space=pl.ANY),
        ],
        out_specs=pl.BlockSpec(memory_space=pl.ANY),
        scratch_shapes=[
            pltpu.VMEM((BATCH, 8, D), jnp.float32),
            pltpu.VMEM((BATCH, STAGE, D), jnp.float32),
            pltpu.SemaphoreType.DMA,
            pltpu.SemaphoreType.DMA,
            pltpu.SemaphoreType.DMA,
            pltpu.SemaphoreType.DMA,
        ],
    )(seq_a, seq_b)

    mask, ids = pl.pallas_call(
        _mask_body,
        grid=(NBLK,),
        out_shape=[
            jax.ShapeDtypeStruct((1, 1, PADDED, PADDED), jnp.bool_),
            jax.ShapeDtypeStruct((1, PADDED), jnp.int32),
        ],
        out_specs=[
            pl.BlockSpec((1, 1, BLK, PADDED), lambda i: (0, 0, i, 0)),
            pl.BlockSpec((1, PADDED), lambda i: (0, 0)),
        ],
    )()

    return packed, mask, ids.reshape(PADDED)


# speedup vs baseline: 14.9431x; 14.0946x over previous
import jax
import jax.numpy as jnp
from jax import lax
from jax.experimental import pallas as pl
from jax.experimental.pallas import tpu as pltpu

LEN_A = 2000
LEN_B = 1500
SEQ = LEN_A + LEN_B
PADDED = 3584
D = 1024
BATCH = 8
BLK = 128
NBLK = PADDED // BLK
NBLK_A = (LEN_A + BLK - 1) // BLK
NBLK_B = (LEN_B + BLK - 1) // BLK
SH = (NBLK_A * BLK - LEN_A) % BLK


def _pack_body(a_ref, b_ref, out_ref, carry_ref):
    k = pl.program_id(1)
    row = k * BLK + lax.broadcasted_iota(jnp.int32, (BLK, D), 0)
    a_val = a_ref[0]
    b_new = b_ref[0]
    b_val = jnp.concatenate([carry_ref[SH:BLK, :], b_new[0:SH, :]], axis=0)
    out = jnp.where(row < LEN_A, a_val,
                    jnp.where(row < SEQ, b_val, jnp.float32(0.0)))
    out_ref[0] = out
    carry_ref[...] = b_new


def _mask_body(mask_ref, ids_ref):
    i = pl.program_id(0)
    row = i * BLK + lax.broadcasted_iota(jnp.int32, (BLK, PADDED), 0)
    col = lax.broadcasted_iota(jnp.int32, (BLK, PADDED), 1)
    in_a = (row < LEN_A) & (col < LEN_A)
    in_b = (row >= LEN_A) & (row < SEQ) & (col >= LEN_A) & (col < SEQ)
    m = (col <= row) & (in_a | in_b)
    mask_ref[...] = m.reshape(1, 1, BLK, PADDED)

    @pl.when(i == 0)
    def _():
        c = lax.broadcasted_iota(jnp.int32, (1, PADDED), 1)
        ids_ref[...] = jnp.where(c < LEN_A, 0, jnp.where(c < SEQ, 1, -1))


@jax.jit
def kernel(seq_a, seq_b):
    packed = pl.pallas_call(
        _pack_body,
        grid=(BATCH, NBLK),
        out_shape=jax.ShapeDtypeStruct((BATCH, PADDED, D), jnp.float32),
        in_specs=[
            pl.BlockSpec((1, BLK, D),
                         lambda i, k: (i, jnp.minimum(k, NBLK_A - 1), 0)),
            pl.BlockSpec((1, BLK, D),
                         lambda i, k: (i, jnp.clip(k - NBLK_A + 1, 0,
                                                   NBLK_B - 1), 0)),
        ],
        out_specs=pl.BlockSpec((1, BLK, D), lambda i, k: (i, k, 0)),
        scratch_shapes=[pltpu.VMEM((BLK, D), jnp.float32)],
    )(seq_a, seq_b)

    mask, ids = pl.pallas_call(
        _mask_body,
        grid=(NBLK,),
        out_shape=[
            jax.ShapeDtypeStruct((1, 1, PADDED, PADDED), jnp.bool_),
            jax.ShapeDtypeStruct((1, PADDED), jnp.int32),
        ],
        out_specs=[
            pl.BlockSpec((1, 1, BLK, PADDED), lambda i: (0, 0, i, 0)),
            pl.BlockSpec((1, PADDED), lambda i: (0, 0)),
        ],
    )()

    return packed, mask, ids.reshape(PADDED)


# speedup vs baseline: 24.2376x; 1.6220x over previous
import jax
import jax.numpy as jnp
from jax import lax
from jax.experimental import pallas as pl
from jax.experimental.pallas import tpu as pltpu

LEN_A = 2000
LEN_B = 1500
SEQ = LEN_A + LEN_B
PADDED = 3584
D = 1024
BATCH = 8
BLK = 128
NBLK = PADDED // BLK

B_MAIN = (LEN_B // 8) * 8
B_TAIL = LEN_B - B_MAIN
STAGE = PADDED - LEN_A - B_MAIN

CH = 400
NSLOT = 8
LOOKAHEAD = 4


def _chunks():
    table = []
    for i in range(BATCH):
        for off in range(0, LEN_A, CH):
            n = min(CH, LEN_A - off)
            table.append((0, i, off, off, n))
        for off in range(0, B_MAIN, CH):
            n = min(CH, B_MAIN - off)
            table.append((1, i, off, LEN_A + off, n))
    return table


_TABLE = _chunks()


def _pack_body(a_ref, b_ref, out_ref, slots, tail_vm, stage_vm, in_sems,
               out_sems, sem_t, sem_s):
    srcs = (a_ref, b_ref)

    t_start = pl.multiple_of(B_MAIN + pl.program_id(0), 8)
    cp_t = pltpu.make_async_copy(b_ref.at[:, pl.ds(t_start, 8), :], tail_vm,
                                 sem_t)
    cp_t.start()

    n = len(_TABLE)
    in_cp = [None] * n
    out_cp = [None] * n

    def make_in(idx):
        src_id, i, s_off, _, rows = _TABLE[idx]
        s = idx % NSLOT
        return pltpu.make_async_copy(
            srcs[src_id].at[i, pl.ds(s_off, rows), :],
            slots.at[s, pl.ds(0, rows), :], in_sems.at[s])

    def make_out(idx):
        _, i, _, d_off, rows = _TABLE[idx]
        s = idx % NSLOT
        return pltpu.make_async_copy(
            slots.at[s, pl.ds(0, rows), :],
            out_ref.at[i, pl.ds(d_off, rows), :], out_sems.at[s])

    for idx in range(n + LOOKAHEAD):
        if idx < n:
            if idx >= NSLOT:
                out_cp[idx - NSLOT].wait()
            in_cp[idx] = make_in(idx)
            in_cp[idx].start()
        j = idx - LOOKAHEAD
        if 0 <= j < n:
            in_cp[j].wait()
            out_cp[j] = make_out(j)
            out_cp[j].start()

    cp_t.wait()
    stage_vm[...] = jnp.zeros((BATCH, STAGE, D), jnp.float32)
    stage_vm[:, 0:B_TAIL, :] = tail_vm[:, 0:B_TAIL, :]
    cp_s = pltpu.make_async_copy(stage_vm,
                                 out_ref.at[:, LEN_A + B_MAIN:PADDED, :],
                                 sem_s)
    cp_s.start()
    cp_s.wait()

    for j in range(max(n - NSLOT, 0), n):
        out_cp[j].wait()


def _mask_body(mask_ref, ids_ref):
    i = pl.program_id(0)
    row = i * BLK + lax.broadcasted_iota(jnp.int32, (BLK, PADDED), 0)
    col = lax.broadcasted_iota(jnp.int32, (BLK, PADDED), 1)
    in_a = (row < LEN_A) & (col < LEN_A)
    in_b = (row >= LEN_A) & (row < SEQ) & (col >= LEN_A) & (col < SEQ)
    m = (col <= row) & (in_a | in_b)
    mask_ref[...] = m.reshape(1, 1, BLK, PADDED)

    @pl.when(i == 0)
    def _():
        c = lax.broadcasted_iota(jnp.int32, (1, PADDED), 1)
        ids_ref[...] = jnp.where(c < LEN_A, 0, jnp.where(c < SEQ, 1, -1))


@jax.jit
def kernel(seq_a, seq_b):
    packed = pl.pallas_call(
        _pack_body,
        grid=(1,),
        out_shape=jax.ShapeDtypeStruct((BATCH, PADDED, D), jnp.float32),
        in_specs=[
            pl.BlockSpec(memory_space=pl.ANY),
            pl.BlockSpec(memory_space=pl.ANY),
        ],
        out_specs=pl.BlockSpec(memory_space=pl.ANY),
        scratch_shapes=[
            pltpu.VMEM((NSLOT, CH, D), jnp.float32),
            pltpu.VMEM((BATCH, 8, D), jnp.float32),
            pltpu.VMEM((BATCH, STAGE, D), jnp.float32),
            pltpu.SemaphoreType.DMA((NSLOT,)),
            pltpu.SemaphoreType.DMA((NSLOT,)),
            pltpu.SemaphoreType.DMA,
            pltpu.SemaphoreType.DMA,
        ],
    )(seq_a, seq_b)

    mask, ids = pl.pallas_call(
        _mask_body,
        grid=(NBLK,),
        out_shape=[
            jax.ShapeDtypeStruct((1, 1, PADDED, PADDED), jnp.bool_),
            jax.ShapeDtypeStruct((1, PADDED), jnp.int32),
        ],
        out_specs=[
            pl.BlockSpec((1, 1, BLK, PADDED), lambda i: (0, 0, i, 0)),
            pl.BlockSpec((1, PADDED), lambda i: (0, 0)),
        ],
    )()

    return packed, mask, ids.reshape(PADDED)
